# Initial kernel scaffold; baseline (speedup 1.0000x reference)
import jax
import jax.numpy as jnp
from jax.experimental import pallas as pl


def kernel(x, adj, W, bias):
    raise NotImplementedError("write your pallas kernel here")


# speedup vs baseline: 3.1049x; 3.1049x over previous
import functools

import jax
import jax.numpy as jnp
from jax import lax
from jax.experimental import pallas as pl
from jax.experimental.pallas import tpu as pltpu
from jax.experimental.pallas import tpu_sc as plsc

N = 10000
NPAD = 10240
F = 128
NNZ = 320000
NC = 2
NS = 16
G = 8
MROW = NPAD // NS
CW = 125
IB = 16
PCH = NNZ // NC // CW
PNB = PCH // IB
HCH = NNZ // NS // CW
HNB = HCH // IB

_MESH = plsc.VectorSubcoreMesh(core_axis_name="c", subcore_axis_name="s")
_SC_PARAMS = pltpu.CompilerParams(use_tc_tiling_on_sc=False)


def _matmul_body(x_ref, w_ref, o_ref):
    o_ref[...] = lax.dot_general(
        x_ref[...], w_ref[...], (((1,), (1,)), ((), ())),
        preferred_element_type=jnp.float32)


def _matmul(x, w):
    return pl.pallas_call(
        _matmul_body,
        out_shape=jax.ShapeDtypeStruct((N, F), jnp.float32),
    )(x, w)


_RB = 128


def _combine_mid_body(p_ref, hc_ref, o_ref):
    cnt = jnp.sum(hc_ref[...], axis=0)
    recip = jnp.where(cnt > 0, 1.0 / cnt, 0.0)
    o_ref[...] = (p_ref[0] + p_ref[1]) * recip[None]


def _combine_mid(p, hc):
    return pl.pallas_call(
        _combine_mid_body,
        grid=(MROW // _RB,),
        in_specs=[
            pl.BlockSpec((NC, NS, _RB, F), lambda i: (0, 0, i, 0)),
            pl.BlockSpec((NS, _RB, F), lambda i: (0, i, 0)),
        ],
        out_specs=pl.BlockSpec((NS, _RB, F), lambda i: (0, i, 0)),
        out_shape=jax.ShapeDtypeStruct((NS, MROW, F), jnp.float32),
    )(p, hc)


def _combine_out_body(q_ref, hr_ref, b_ref, o_ref):
    cnt = jnp.sum(hr_ref[...], axis=0)
    recip = jnp.where(cnt > 0, 1.0 / cnt, 0.0)
    o_ref[...] = jnp.maximum(
        (q_ref[0] + q_ref[1]) * recip[None] + b_ref[...], 0.0)


def _combine_out(q, hr, bias_r):
    return pl.pallas_call(
        _combine_out_body,
        grid=(MROW // _RB,),
        in_specs=[
            pl.BlockSpec((NC, NS, _RB, F), lambda i: (0, 0, i, 0)),
            pl.BlockSpec((NS, _RB, F), lambda i: (0, i, 0)),
            pl.BlockSpec((NS, 1, F), lambda i: (0, 0, 0)),
        ],
        out_specs=pl.BlockSpec((NS, _RB, F), lambda i: (0, i, 0)),
        out_shape=jax.ShapeDtypeStruct((NS, MROW, F), jnp.float32),
    )(q, hr, bias_r)


def _scatter_pass_body(tab, src_idx, dst_idx, zeros_g,
                       p_out,
                       iv_s, iv_d, gbuf, acc):
    c = lax.axis_index("c")
    s = lax.axis_index("s")
    pltpu.sync_copy(zeros_g, acc.at[s])
    tab_g = tab.at[s]
    acc_g = acc.at[s]

    def block(b, carry):
        pltpu.sync_copy(src_idx.at[c, pl.ds(b * IB, IB)], iv_s)
        pltpu.sync_copy(dst_idx.at[c, pl.ds(b * IB, IB)], iv_d)

        def chunk(k, carry2):
            pltpu.sync_copy(tab_g.at[iv_s.at[k]], gbuf)
            pltpu.sync_copy(gbuf, acc_g.at[iv_d.at[k]], add=True)
            return carry2

        lax.fori_loop(0, IB, chunk, 0)
        return carry

    lax.fori_loop(0, PNB, block, 0)
    pltpu.sync_copy(acc.at[s], p_out.at[c, s])


_scatter_pass = functools.partial(
    pl.kernel,
    out_type=jax.ShapeDtypeStruct((NC, NS, NPAD, G), jnp.float32),
    mesh=_MESH,
    compiler_params=_SC_PARAMS,
    scratch_types=[
        pltpu.VMEM((IB, CW), jnp.int32),
        pltpu.VMEM((IB, CW), jnp.int32),
        pltpu.VMEM((CW, G), jnp.float32),
        pltpu.VMEM_SHARED((NS, NPAD, G), jnp.float32),
    ],
)(_scatter_pass_body)


def _hist_body(row_idx, col_idx, zeros_g, ones_g,
               hr_out, hc_out,
               iv, obuf, acc):
    c = lax.axis_index("c")
    s = lax.axis_index("s")
    pltpu.sync_copy(zeros_g, acc.at[s])
    pltpu.sync_copy(ones_g, obuf)
    acc_g = acc.at[s]

    def mk_block(idx):
        def block(b, carry):
            pltpu.sync_copy(idx.at[s, pl.ds(b * IB, IB)], iv)

            def chunk(k, carry2):
                pltpu.sync_copy(obuf, acc_g.at[iv.at[k]], add=True)
                return carry2

            lax.fori_loop(0, IB, chunk, 0)
            return carry
        return block

    @pl.when(c == 0)
    def _():
        lax.fori_loop(0, HNB, mk_block(row_idx), 0)
        pltpu.sync_copy(acc.at[s], hr_out.at[s])

    @pl.when(c == 1)
    def _():
        lax.fori_loop(0, HNB, mk_block(col_idx), 0)
        pltpu.sync_copy(acc.at[s], hc_out.at[s])


_hist = functools.partial(
    pl.kernel,
    out_type=(
        jax.ShapeDtypeStruct((NS, NPAD, G), jnp.float32),
        jax.ShapeDtypeStruct((NS, NPAD, G), jnp.float32),
    ),
    mesh=_MESH,
    compiler_params=_SC_PARAMS,
    scratch_types=[
        pltpu.VMEM((IB, CW), jnp.int32),
        pltpu.VMEM((CW, G), jnp.float32),
        pltpu.VMEM_SHARED((NS, NPAD, G), jnp.float32),
    ],
)(_hist_body)


def kernel(x, adj, W, bias):
    row = adj[0].astype(jnp.int32)
    col = adj[1].astype(jnp.int32)
    row_p = row.reshape(NC, PCH, CW)
    col_p = col.reshape(NC, PCH, CW)
    row_h = row.reshape(NS, HCH, CW)
    col_h = col.reshape(NS, HCH, CW)
    zeros_g = jnp.zeros((NPAD, G), jnp.float32)
    ones_g = jnp.ones((CW, G), jnp.float32)
    bias_r = jnp.tile(bias.reshape(NS, 1, G), (1, NS, 1)).reshape(NS, 1, F)

    hr, hc = _hist(row_h, col_h, zeros_g, ones_g)
    xt = _matmul(x, W)
    tab1 = xt.reshape(N, NS, G).transpose(1, 0, 2)
    p = _scatter_pass(tab1, row_p, col_p, zeros_g)
    e_r = _combine_mid(p.reshape(NC, NS, MROW, F), hc.reshape(NS, MROW, F))
    q = _scatter_pass(e_r.reshape(NS, NPAD, G), col_p, row_p, zeros_g)
    o_r = _combine_out(q.reshape(NC, NS, MROW, F), hr.reshape(NS, MROW, F),
                       bias_r)
    return (o_r.reshape(NS, NPAD, G)[:, :N]
            .transpose(1, 0, 2).reshape(N, F))


# speedup vs baseline: 5.6330x; 1.8142x over previous
import functools

import jax
import jax.numpy as jnp
from jax import lax
from jax.experimental import pallas as pl
from jax.experimental.pallas import tpu as pltpu
from jax.experimental.pallas import tpu_sc as plsc

N = 10000
NPAD = 10240
F = 128
NNZ = 320000
NC = 2
NS = 16
G = 8
MROW = NPAD // NS
CW = 125
IB = 16
PCH = NNZ // NC // CW
PNB = PCH // IB
HCH = NNZ // NS // CW
HNB = HCH // IB

_MESH = plsc.VectorSubcoreMesh(core_axis_name="c", subcore_axis_name="s")
_SC_PARAMS = pltpu.CompilerParams(use_tc_tiling_on_sc=False)


def _matmul_body(x_ref, w_ref, o_ref):
    o_ref[...] = lax.dot_general(
        x_ref[...], w_ref[...], (((1,), (1,)), ((), ())),
        preferred_element_type=jnp.float32)


def _matmul(x, w):
    return pl.pallas_call(
        _matmul_body,
        out_shape=jax.ShapeDtypeStruct((N, F), jnp.float32),
    )(x, w)


_RB = 128


def _combine_mid_body(p_ref, hc_ref, o_ref):
    cnt = jnp.sum(hc_ref[...], axis=0)
    recip = jnp.where(cnt > 0, 1.0 / cnt, 0.0)
    o_ref[...] = (p_ref[0] + p_ref[1]) * recip[None]


def _combine_mid(p, hc):
    return pl.pallas_call(
        _combine_mid_body,
        grid=(MROW // _RB,),
        in_specs=[
            pl.BlockSpec((NC, NS, _RB, F), lambda i: (0, 0, i, 0)),
            pl.BlockSpec((NS, _RB, F), lambda i: (0, i, 0)),
        ],
        out_specs=pl.BlockSpec((NS, _RB, F), lambda i: (0, i, 0)),
        out_shape=jax.ShapeDtypeStruct((NS, MROW, F), jnp.float32),
    )(p, hc)


def _combine_out_body(q_ref, hr_ref, b_ref, o_ref):
    cnt = jnp.sum(hr_ref[...], axis=0)
    recip = jnp.where(cnt > 0, 1.0 / cnt, 0.0)
    o_ref[...] = jnp.maximum(
        (q_ref[0] + q_ref[1]) * recip[None] + b_ref[...], 0.0)


def _combine_out(q, hr, bias_r):
    return pl.pallas_call(
        _combine_out_body,
        grid=(MROW // _RB,),
        in_specs=[
            pl.BlockSpec((NC, NS, _RB, F), lambda i: (0, 0, i, 0)),
            pl.BlockSpec((NS, _RB, F), lambda i: (0, i, 0)),
            pl.BlockSpec((NS, 1, F), lambda i: (0, 0, 0)),
        ],
        out_specs=pl.BlockSpec((NS, _RB, F), lambda i: (0, i, 0)),
        out_shape=jax.ShapeDtypeStruct((NS, MROW, F), jnp.float32),
    )(q, hr, bias_r)


_NBUF = 4


def _scatter_pass_body(tab, src_idx, dst_idx, zeros_g,
                       p_out,
                       iv_s, iv_d, gbuf, acc, *sems):
    c = lax.axis_index("c")
    s = lax.axis_index("s")
    pltpu.sync_copy(zeros_g, acc.at[s])
    tab_g = tab.at[s]
    acc_g = acc.at[s]

    def block(b, carry):
        pltpu.sync_copy(src_idx.at[c, pl.ds(b * IB, IB)], iv_s)
        pltpu.sync_copy(dst_idx.at[c, pl.ds(b * IB, IB)], iv_d)
        descs = [
            pltpu.async_copy(tab_g.at[iv_s.at[j]], gbuf.at[j], sems[j])
            for j in range(_NBUF)
        ]
        for k in range(IB):
            j = k % _NBUF
            descs[j].wait()
            pltpu.sync_copy(gbuf.at[j], acc_g.at[iv_d.at[k]], add=True)
            if k + _NBUF < IB:
                descs[j] = pltpu.async_copy(
                    tab_g.at[iv_s.at[k + _NBUF]], gbuf.at[j], sems[j])
        return carry

    lax.fori_loop(0, PNB, block, 0)
    pltpu.sync_copy(acc.at[s], p_out.at[c, s])


_scatter_pass = functools.partial(
    pl.kernel,
    out_type=jax.ShapeDtypeStruct((NC, NS, NPAD, G), jnp.float32),
    mesh=_MESH,
    compiler_params=_SC_PARAMS,
    scratch_types=[
        pltpu.VMEM((IB, CW), jnp.int32),
        pltpu.VMEM((IB, CW), jnp.int32),
        pltpu.VMEM((_NBUF, CW, G), jnp.float32),
        pltpu.VMEM_SHARED((NS, NPAD, G), jnp.float32),
    ] + [pltpu.SemaphoreType.DMA] * _NBUF,
)(_scatter_pass_body)


def _hist_body(row_idx, col_idx, zeros_g, ones_g,
               hr_out, hc_out,
               iv, obuf, acc):
    c = lax.axis_index("c")
    s = lax.axis_index("s")
    pltpu.sync_copy(zeros_g, acc.at[s])
    pltpu.sync_copy(ones_g, obuf)
    acc_g = acc.at[s]

    def mk_block(idx):
        def block(b, carry):
            pltpu.sync_copy(idx.at[s, pl.ds(b * IB, IB)], iv)

            def chunk(k, carry2):
                pltpu.sync_copy(obuf, acc_g.at[iv.at[k]], add=True)
                return carry2

            lax.fori_loop(0, IB, chunk, 0)
            return carry
        return block

    @pl.when(c == 0)
    def _():
        lax.fori_loop(0, HNB, mk_block(row_idx), 0)
        pltpu.sync_copy(acc.at[s], hr_out.at[s])

    @pl.when(c == 1)
    def _():
        lax.fori_loop(0, HNB, mk_block(col_idx), 0)
        pltpu.sync_copy(acc.at[s], hc_out.at[s])


_hist = functools.partial(
    pl.kernel,
    out_type=(
        jax.ShapeDtypeStruct((NS, NPAD, G), jnp.float32),
        jax.ShapeDtypeStruct((NS, NPAD, G), jnp.float32),
    ),
    mesh=_MESH,
    compiler_params=_SC_PARAMS,
    scratch_types=[
        pltpu.VMEM((IB, CW), jnp.int32),
        pltpu.VMEM((CW, G), jnp.float32),
        pltpu.VMEM_SHARED((NS, NPAD, G), jnp.float32),
    ],
)(_hist_body)


def kernel(x, adj, W, bias):
    row = adj[0].astype(jnp.int32)
    col = adj[1].astype(jnp.int32)
    row_p = row.reshape(NC, PCH, CW)
    col_p = col.reshape(NC, PCH, CW)
    row_h = row.reshape(NS, HCH, CW)
    col_h = col.reshape(NS, HCH, CW)
    zeros_g = jnp.zeros((NPAD, G), jnp.float32)
    ones_g = jnp.ones((CW, G), jnp.float32)
    bias_r = jnp.tile(bias.reshape(NS, 1, G), (1, NS, 1)).reshape(NS, 1, F)

    hr, hc = _hist(row_h, col_h, zeros_g, ones_g)
    xt = _matmul(x, W)
    tab1 = xt.reshape(N, NS, G).transpose(1, 0, 2)
    p = _scatter_pass(tab1, row_p, col_p, zeros_g)
    e_r = _combine_mid(p.reshape(NC, NS, MROW, F), hc.reshape(NS, MROW, F))
    q = _scatter_pass(e_r.reshape(NS, NPAD, G), col_p, row_p, zeros_g)
    o_r = _combine_out(q.reshape(NC, NS, MROW, F), hr.reshape(NS, MROW, F),
                       bias_r)
    return (o_r.reshape(NS, NPAD, G)[:, :N]
            .transpose(1, 0, 2).reshape(N, F))


# speedup vs baseline: 5.8884x; 1.0453x over previous
import functools

import jax
import jax.numpy as jnp
from jax import lax
from jax.experimental import pallas as pl
from jax.experimental.pallas import tpu as pltpu
from jax.experimental.pallas import tpu_sc as plsc

N = 10000
NPAD = 10240
F = 128
NNZ = 320000
NC = 2
NS = 16
G = 8
CW = 125
IB = 16
PCH = NNZ // NC // CW
PNB = PCH // IB
HCH = NNZ // NS // CW
HNB = HCH // IB
RPT = NPAD // (NC * NS)
RPS = NPAD // NS
GW = NPAD * G

_MESH = plsc.VectorSubcoreMesh(core_axis_name="c", subcore_axis_name="s")
_SC_PARAMS = pltpu.CompilerParams(use_tc_tiling_on_sc=False)


def _lane_pattern(g):
    i = lax.iota(jnp.int32, 16)
    return (i // G) * F + G * g + (i % G)


def _matmul_body(x_ref, w_ref, o_ref):
    o_ref[...] = lax.dot_general(
        x_ref[...], w_ref[...], (((1,), (1,)), ((), ())),
        preferred_element_type=jnp.float32)


def _matmul(x, w):
    return pl.pallas_call(
        _matmul_body,
        out_shape=jax.ShapeDtypeStruct((NPAD, F), jnp.float32),
    )(x, w)


def _split_body(xt2, tab3, v2d):
    c = lax.axis_index("c")
    s = lax.axis_index("s")
    r0 = c * (NPAD // NC)
    pltpu.sync_copy(xt2.at[pl.ds(r0, NPAD // NC), pl.ds(G * s, G)], v2d)
    pltpu.sync_copy(v2d, tab3.at[s, pl.ds(r0, NPAD // NC)])


_split = functools.partial(
    pl.kernel,
    out_type=jax.ShapeDtypeStruct((NS, NPAD, G), jnp.float32),
    mesh=_MESH,
    compiler_params=_SC_PARAMS,
    scratch_types=[
        pltpu.VMEM((NPAD // NC, G), jnp.float32),
    ],
)(_split_body)


def _hist_body(row_idx, col_idx, zeros_g, ones_g,
               part_out,
               iv, obuf, acc):
    c = lax.axis_index("c")
    s = lax.axis_index("s")
    pltpu.sync_copy(zeros_g, acc.at[s])
    pltpu.sync_copy(ones_g, obuf)
    acc_g = acc.at[s]

    def mk_block(idx):
        def block(b, carry):
            pltpu.sync_copy(idx.at[s, pl.ds(b * IB, IB)], iv)

            def chunk(k, carry2):
                pltpu.sync_copy(obuf, acc_g.at[iv.at[k]], add=True)
                return carry2

            lax.fori_loop(0, IB, chunk, 0)
            return carry
        return block

    @pl.when(c == 0)
    def _():
        lax.fori_loop(0, HNB, mk_block(row_idx), 0)

    @pl.when(c == 1)
    def _():
        lax.fori_loop(0, HNB, mk_block(col_idx), 0)

    pltpu.sync_copy(acc.at[s], part_out.at[c, s])


_hist = functools.partial(
    pl.kernel,
    out_type=jax.ShapeDtypeStruct((NC, NS, NPAD, G), jnp.float32),
    mesh=_MESH,
    compiler_params=_SC_PARAMS,
    scratch_types=[
        pltpu.VMEM((IB, CW), jnp.int32),
        pltpu.VMEM((CW, G), jnp.float32),
        pltpu.VMEM_SHARED((NS, NPAD, G), jnp.float32),
    ],
)(_hist_body)


_HW = GW // NS


def _hist_reduce_body(part_f, rr_out, rc_out, abuf, rbuf):
    c = lax.axis_index("c")
    s = lax.axis_index("s")
    off = s * _HW
    pltpu.sync_copy(part_f.at[c, 0, pl.ds(off, _HW)], abuf)

    def add_tile(t, carry):
        pltpu.sync_copy(part_f.at[c, t, pl.ds(off, _HW)], rbuf)

        def vec(i, carry2):
            sl = pl.ds(16 * i, 16)
            abuf[sl] = abuf[sl] + rbuf[sl]
            return carry2

        lax.fori_loop(0, _HW // 16, vec, 0)
        return carry

    lax.fori_loop(1, NS, add_tile, 0)

    def recip(i, carry):
        sl = pl.ds(16 * i, 16)
        v = abuf[sl]
        abuf[sl] = jnp.where(v > 0, 1.0 / v, 0.0)
        return carry

    lax.fori_loop(0, _HW // 16, recip, 0)

    @pl.when(c == 0)
    def _():
        pltpu.sync_copy(abuf, rr_out.at[pl.ds(off, _HW)])

    @pl.when(c == 1)
    def _():
        pltpu.sync_copy(abuf, rc_out.at[pl.ds(off, _HW)])


_hist_reduce = functools.partial(
    pl.kernel,
    out_type=(
        jax.ShapeDtypeStruct((GW,), jnp.float32),
        jax.ShapeDtypeStruct((GW,), jnp.float32),
    ),
    mesh=_MESH,
    compiler_params=_SC_PARAMS,
    scratch_types=[
        pltpu.VMEM((_HW,), jnp.float32),
        pltpu.VMEM((_HW,), jnp.float32),
    ],
)(_hist_reduce_body)


_NBUF = 4


def _scatter_pass_body(tab, src_idx, dst_idx, zeros_g,
                       p_out,
                       iv_s, iv_d, gbuf, acc, *sems):
    c = lax.axis_index("c")
    s = lax.axis_index("s")
    pltpu.sync_copy(zeros_g, acc.at[s])
    tab_g = tab.at[s]
    acc_g = acc.at[s]

    def block(b, carry):
        pltpu.sync_copy(src_idx.at[c, pl.ds(b * IB, IB)], iv_s)
        pltpu.sync_copy(dst_idx.at[c, pl.ds(b * IB, IB)], iv_d)
        descs = [
            pltpu.async_copy(tab_g.at[iv_s.at[j]], gbuf.at[j], sems[j])
            for j in range(_NBUF)
        ]
        for k in range(IB):
            j = k % _NBUF
            descs[j].wait()
            pltpu.sync_copy(gbuf.at[j], acc_g.at[iv_d.at[k]], add=True)
            if k + _NBUF < IB:
                descs[j] = pltpu.async_copy(
                    tab_g.at[iv_s.at[k + _NBUF]], gbuf.at[j], sems[j])
        return carry

    lax.fori_loop(0, PNB, block, 0)
    pltpu.sync_copy(acc.at[s], p_out.at[c, s])


_scatter_pass = functools.partial(
    pl.kernel,
    out_type=jax.ShapeDtypeStruct((NC, NS, NPAD, G), jnp.float32),
    mesh=_MESH,
    compiler_params=_SC_PARAMS,
    scratch_types=[
        pltpu.VMEM((IB, CW), jnp.int32),
        pltpu.VMEM((IB, CW), jnp.int32),
        pltpu.VMEM((_NBUF, CW, G), jnp.float32),
        pltpu.VMEM_SHARED((NS, NPAD, G), jnp.float32),
    ] + [pltpu.SemaphoreType.DMA] * _NBUF,
)(_scatter_pass_body)


_MCH = 4
_MW = NPAD // NC * G // _MCH


def _mid_body(p_f, recip_c, e_f, b0, b1, br, bo):
    c = lax.axis_index("c")
    s = lax.axis_index("s")
    base = c * (NPAD // NC) * G

    def sub(m, carry):
        off = base + m * _MW
        pltpu.sync_copy(p_f.at[0, s, pl.ds(off, _MW)], b0)
        pltpu.sync_copy(p_f.at[1, s, pl.ds(off, _MW)], b1)
        pltpu.sync_copy(recip_c.at[pl.ds(off, _MW)], br)

        def vec(i, carry2):
            sl = pl.ds(16 * i, 16)
            bo[sl] = (b0[sl] + b1[sl]) * br[sl]
            return carry2

        lax.fori_loop(0, _MW // 16, vec, 0)
        pltpu.sync_copy(bo, e_f.at[s, pl.ds(off, _MW)])
        return carry

    lax.fori_loop(0, _MCH, sub, 0)


_mid = functools.partial(
    pl.kernel,
    out_type=jax.ShapeDtypeStruct((NS, GW), jnp.float32),
    mesh=_MESH,
    compiler_params=_SC_PARAMS,
    scratch_types=[
        pltpu.VMEM((_MW,), jnp.float32),
        pltpu.VMEM((_MW,), jnp.float32),
        pltpu.VMEM((_MW,), jnp.float32),
        pltpu.VMEM((_MW,), jnp.float32),
    ],
)(_mid_body)


def _finishf_body(q_f, recip_r, bias16, osp_f, b0, b1, br, bo, bvec):
    c = lax.axis_index("c")
    s = lax.axis_index("s")
    base = c * (NPAD // NC) * G
    pltpu.sync_copy(bias16.at[s], bvec)

    def sub(m, carry):
        off = base + m * _MW
        pltpu.sync_copy(q_f.at[0, s, pl.ds(off, _MW)], b0)
        pltpu.sync_copy(q_f.at[1, s, pl.ds(off, _MW)], b1)
        pltpu.sync_copy(recip_r.at[pl.ds(off, _MW)], br)
        bv = bvec[pl.ds(0, 16)]

        def vec(i, carry2):
            sl = pl.ds(16 * i, 16)
            bo[sl] = jnp.maximum((b0[sl] + b1[sl]) * br[sl] + bv, 0.0)
            return carry2

        lax.fori_loop(0, _MW // 16, vec, 0)
        pltpu.sync_copy(bo, osp_f.at[s, pl.ds(off, _MW)])
        return carry

    lax.fori_loop(0, _MCH, sub, 0)


_finishf = functools.partial(
    pl.kernel,
    out_type=jax.ShapeDtypeStruct((NS, GW), jnp.float32),
    mesh=_MESH,
    compiler_params=_SC_PARAMS,
    scratch_types=[
        pltpu.VMEM((_MW,), jnp.float32),
        pltpu.VMEM((_MW,), jnp.float32),
        pltpu.VMEM((_MW,), jnp.float32),
        pltpu.VMEM((_MW,), jnp.float32),
        pltpu.VMEM((16,), jnp.float32),
    ],
)(_finishf_body)


_AR = N // NC


def _assemble_body(osp3, out2, v2d):
    c = lax.axis_index("c")
    s = lax.axis_index("s")
    r0 = c * _AR
    pltpu.sync_copy(osp3.at[s, pl.ds(r0, _AR)], v2d)
    pltpu.sync_copy(v2d, out2.at[pl.ds(r0, _AR), pl.ds(G * s, G)])


_assemble = functools.partial(
    pl.kernel,
    out_type=jax.ShapeDtypeStruct((N, F), jnp.float32),
    mesh=_MESH,
    compiler_params=_SC_PARAMS,
    scratch_types=[
        pltpu.VMEM((_AR, G), jnp.float32),
    ],
)(_assemble_body)


def kernel(x, adj, W, bias):
    row = adj[0].astype(jnp.int32)
    col = adj[1].astype(jnp.int32)
    row_p = row.reshape(NC, PCH, CW)
    col_p = col.reshape(NC, PCH, CW)
    row_h = row.reshape(NS, HCH, CW)
    col_h = col.reshape(NS, HCH, CW)
    zeros_g = jnp.zeros((NPAD, G), jnp.float32)
    ones_g = jnp.ones((CW, G), jnp.float32)
    bias16 = jnp.tile(bias.reshape(NS, 1, G), (1, 2, 1)).reshape(NS, 16)
    x_p = jnp.pad(x, ((0, NPAD - N), (0, 0)))

    hp = _hist(row_h, col_h, zeros_g, ones_g)
    recip_row, recip_col = _hist_reduce(hp.reshape(NC, NS, GW))
    xt = _matmul(x_p, W)
    tab1 = _split(xt)
    p = _scatter_pass(tab1, row_p, col_p, zeros_g)
    e_f = _mid(p.reshape(NC, NS, GW), recip_col)
    q = _scatter_pass(e_f.reshape(NS, NPAD, G), col_p, row_p, zeros_g)
    osp = _finishf(q.reshape(NC, NS, GW), recip_row, bias16)
    return _assemble(osp.reshape(NS, NPAD, G))
